# baseline (speedup 1.0000x reference)
import functools

import jax
import jax.numpy as jnp
from jax import lax
from jax.experimental import pallas as pl
from jax.experimental.pallas import tpu as pltpu
from jax.experimental.pallas import tpu_sc as plsc

VOCAB = 1000000
EMBED_DIM = 32
BATCH = 4096
SEQ = 200

NC = 2
NS = 16
NW = NC * NS

BT = BATCH // NW
DT = EMBED_DIM // 8

_mesh = plsc.VectorSubcoreMesh(
    core_axis_name="c", subcore_axis_name="s", num_cores=NC, num_subcores=NS
)


@functools.partial(
    pl.kernel,
    out_type=jax.ShapeDtypeStruct((SEQ, DT, NW, 8, BT), jnp.float32),
    mesh=_mesh,
    scratch_types=[
        pltpu.VMEM((SEQ, BT), jnp.int32),
        pltpu.VMEM((BT, EMBED_DIM), jnp.float32),
        pltpu.VMEM((BT, EMBED_DIM), jnp.float32),
        pltpu.VMEM((EMBED_DIM, BT), jnp.float32),
        pltpu.VMEM((EMBED_DIM, BT), jnp.float32),
        pltpu.SemaphoreType.DMA,
        pltpu.SemaphoreType.DMA,
        pltpu.SemaphoreType.DMA,
        pltpu.SemaphoreType.DMA,
    ],
    compiler_params=pltpu.CompilerParams(
        use_tc_tiling_on_sc=False, needs_layout_passes=False
    ),
)
def _embed_sc(idx_hbm, table_hbm, out_hbm, idx_v, rows_a, rows_b,
              tile_a, tile_b, sem_ga, sem_gb, sem_oa, sem_ob):
    wid = lax.axis_index("s") * NC + lax.axis_index("c")
    pltpu.sync_copy(idx_hbm.at[wid], idx_v)

    def fire_gather(s, rows, sem):
        pltpu.async_copy(table_hbm.at[idx_v.at[s]], rows, sem)

    def drain_gather(rows, sem):
        pltpu.make_async_copy(table_hbm.at[pl.ds(0, BT)], rows, sem).wait()

    def drain_out(tile, sem):
        for _ in range(DT):
            pltpu.make_async_copy(
                out_hbm.at[0, 0, 0], tile.at[pl.ds(0, 8)], sem
            ).wait()

    dlo = lax.iota(jnp.int32, 16)
    dhi = dlo + jnp.full((16,), 16, jnp.int32)

    def transpose(rows, tile):
        for c in range(BT):
            i2 = jnp.full((16,), c, jnp.int32)
            plsc.store_scatter(tile, [dlo, i2], rows[c, pl.ds(0, 16)])
            plsc.store_scatter(tile, [dhi, i2], rows[c, pl.ds(16, 16)])

    fire_gather(0, rows_a, sem_ga)

    @pl.loop(0, SEQ, step=2)
    def _pair(s):
        for off, rows, semg, tile, semo, orows, osemg in (
            (0, rows_a, sem_ga, tile_a, sem_oa, rows_b, sem_gb),
            (1, rows_b, sem_gb, tile_b, sem_ob, rows_a, sem_ga),
        ):
            cur = s + off

            @pl.when(cur + 1 < SEQ)
            def _():
                fire_gather(cur + 1, orows, osemg)

            drain_gather(rows, semg)

            @pl.when(cur >= 2)
            def _():
                drain_out(tile, semo)

            transpose(rows, tile)
            for dt in range(DT):
                pltpu.async_copy(
                    tile.at[pl.ds(dt * 8, 8)], out_hbm.at[cur, dt, wid], semo
                )

    drain_out(tile_a, sem_oa)
    drain_out(tile_b, sem_ob)


def kernel(x, table):
    idx = x.astype(jnp.int32).reshape(NW, BT, SEQ).transpose(0, 2, 1)
    out5 = _embed_sc(idx, table)
    return out5.transpose(2, 4, 0, 1, 3).reshape(BATCH, SEQ, EMBED_DIM)


# speedup vs baseline: 1.1701x; 1.1701x over previous
import functools

import jax
import jax.numpy as jnp
from jax import lax
from jax.experimental import pallas as pl
from jax.experimental.pallas import tpu as pltpu
from jax.experimental.pallas import tpu_sc as plsc

VOCAB = 1000000
EMBED_DIM = 32
BATCH = 4096
SEQ = 200

NC = 2
NS = 16
NW = NC * NS

BT = BATCH // NW
DT = EMBED_DIM // 8

_mesh = plsc.VectorSubcoreMesh(
    core_axis_name="c", subcore_axis_name="s", num_cores=NC, num_subcores=NS
)


@functools.partial(
    pl.kernel,
    out_type=jax.ShapeDtypeStruct((SEQ, DT, NW, 8, BT), jnp.float32),
    mesh=_mesh,
    scratch_types=[
        pltpu.VMEM((SEQ, BT), jnp.int32),
        pltpu.VMEM((BT, EMBED_DIM), jnp.float32),
        pltpu.VMEM((BT, EMBED_DIM), jnp.float32),
        pltpu.VMEM((EMBED_DIM, BT), jnp.float32),
        pltpu.VMEM((EMBED_DIM, BT), jnp.float32),
        pltpu.SemaphoreType.DMA,
        pltpu.SemaphoreType.DMA,
        pltpu.SemaphoreType.DMA,
        pltpu.SemaphoreType.DMA,
    ],
    compiler_params=pltpu.CompilerParams(
        use_tc_tiling_on_sc=False, needs_layout_passes=False
    ),
)
def _embed_sc(idx_hbm, table_hbm, out_hbm, idx_v, rows_a, rows_b,
              tile_a, tile_b, sem_ga, sem_gb, sem_oa, sem_ob):
    wid = lax.axis_index("s") * NC + lax.axis_index("c")
    pltpu.sync_copy(idx_hbm.at[wid], idx_v)

    def fire_gather(s, rows, sem):
        pltpu.async_copy(table_hbm.at[idx_v.at[s]], rows, sem)

    def drain_gather(rows, sem):
        pltpu.make_async_copy(table_hbm.at[pl.ds(0, BT)], rows, sem).wait()

    def drain_out(tile, sem):
        for _ in range(DT):
            pltpu.make_async_copy(
                out_hbm.at[0, 0, 0], tile.at[pl.ds(0, 8)], sem
            ).wait()

    dlo = lax.iota(jnp.int32, 16)
    dhi = dlo + jnp.full((16,), 16, jnp.int32)

    def transpose(rows, tile):
        @plsc.parallel_loop(0, BT, unroll=8)
        def _(c):
            i2 = jnp.full((16,), 1, jnp.int32) * c
            plsc.store_scatter(tile, [dlo, i2], rows[c, pl.ds(0, 16)])
            plsc.store_scatter(tile, [dhi, i2], rows[c, pl.ds(16, 16)])

    fire_gather(0, rows_a, sem_ga)

    @pl.loop(0, SEQ, step=2)
    def _pair(s):
        for off, rows, semg, tile, semo, orows, osemg in (
            (0, rows_a, sem_ga, tile_a, sem_oa, rows_b, sem_gb),
            (1, rows_b, sem_gb, tile_b, sem_ob, rows_a, sem_ga),
        ):
            cur = s + off

            @pl.when(cur + 1 < SEQ)
            def _():
                fire_gather(cur + 1, orows, osemg)

            drain_gather(rows, semg)

            @pl.when(cur >= 2)
            def _():
                drain_out(tile, semo)

            transpose(rows, tile)
            for dt in range(DT):
                pltpu.async_copy(
                    tile.at[pl.ds(dt * 8, 8)], out_hbm.at[cur, dt, wid], semo
                )

    drain_out(tile_a, sem_oa)
    drain_out(tile_b, sem_ob)


def kernel(x, table):
    idx = x.astype(jnp.int32).reshape(NW, BT, SEQ).transpose(0, 2, 1)
    out5 = _embed_sc(idx, table)
    return out5.transpose(2, 4, 0, 1, 3).reshape(BATCH, SEQ, EMBED_DIM)


# speedup vs baseline: 1.1743x; 1.0036x over previous
import functools

import jax
import jax.numpy as jnp
from jax import lax
from jax.experimental import pallas as pl
from jax.experimental.pallas import tpu as pltpu
from jax.experimental.pallas import tpu_sc as plsc

VOCAB = 1000000
EMBED_DIM = 32
BATCH = 4096
SEQ = 200

NC = 2
NS = 16
NW = NC * NS

BT = BATCH // NW
DT = EMBED_DIM // 8

_mesh = plsc.VectorSubcoreMesh(
    core_axis_name="c", subcore_axis_name="s", num_cores=NC, num_subcores=NS
)


@functools.partial(
    pl.kernel,
    out_type=jax.ShapeDtypeStruct((SEQ, DT, NW, 8, BT), jnp.float32),
    mesh=_mesh,
    scratch_types=[
        pltpu.VMEM((SEQ, BT), jnp.int32),
        pltpu.VMEM((BT, EMBED_DIM), jnp.float32),
        pltpu.VMEM((BT, EMBED_DIM), jnp.float32),
        pltpu.VMEM((DT, 8, BT), jnp.float32),
        pltpu.VMEM((DT, 8, BT), jnp.float32),
        pltpu.SemaphoreType.DMA,
        pltpu.SemaphoreType.DMA,
        pltpu.SemaphoreType.DMA,
        pltpu.SemaphoreType.DMA,
    ],
    compiler_params=pltpu.CompilerParams(
        use_tc_tiling_on_sc=False, needs_layout_passes=False
    ),
)
def _embed_sc(idx_hbm, table_hbm, out_hbm, idx_v, rows_a, rows_b,
              tile_a, tile_b, sem_ga, sem_gb, sem_oa, sem_ob):
    wid = lax.axis_index("s") * NC + lax.axis_index("c")
    pltpu.sync_copy(idx_hbm.at[wid], idx_v)

    def fire_gather(s, rows, sem):
        pltpu.async_copy(table_hbm.at[idx_v.at[s]], rows, sem)

    def drain_gather(rows, sem):
        pltpu.make_async_copy(table_hbm.at[pl.ds(0, BT)], rows, sem).wait()

    def drain_out(tile, sem):
        pltpu.make_async_copy(out_hbm.at[0, :, 0], tile, sem).wait()

    dlo = lax.iota(jnp.int32, 16)
    dhi = dlo + jnp.full((16,), 16, jnp.int32)
    seven = jnp.full((16,), 7, jnp.int32)
    i0_lo = lax.shift_right_logical(dlo, 3)
    i1_lo = lax.bitwise_and(dlo, seven)
    i0_hi = lax.shift_right_logical(dhi, 3)
    i1_hi = lax.bitwise_and(dhi, seven)
    ones = jnp.full((16,), 1, jnp.int32)
    zeros = jnp.full((16,), 0, jnp.int32)

    def transpose(rows, tile):
        @plsc.parallel_loop(0, BT, unroll=16, carry=zeros)
        def _(c, i2):
            plsc.store_scatter(tile, [i0_lo, i1_lo, i2], rows[c, pl.ds(0, 16)])
            plsc.store_scatter(tile, [i0_hi, i1_hi, i2], rows[c, pl.ds(16, 16)])
            return i2 + ones

    fire_gather(0, rows_a, sem_ga)

    @pl.loop(0, SEQ, step=2)
    def _pair(s):
        for off, rows, semg, tile, semo, orows, osemg in (
            (0, rows_a, sem_ga, tile_a, sem_oa, rows_b, sem_gb),
            (1, rows_b, sem_gb, tile_b, sem_ob, rows_a, sem_ga),
        ):
            cur = s + off

            @pl.when(cur + 1 < SEQ)
            def _():
                fire_gather(cur + 1, orows, osemg)

            drain_gather(rows, semg)

            @pl.when(cur >= 2)
            def _():
                drain_out(tile, semo)

            transpose(rows, tile)
            pltpu.async_copy(tile, out_hbm.at[cur, :, wid], semo)

    drain_out(tile_a, sem_oa)
    drain_out(tile_b, sem_ob)


def kernel(x, table):
    idx = x.astype(jnp.int32).reshape(NW, BT, SEQ).transpose(0, 2, 1)
    out5 = _embed_sc(idx, table)
    return out5.transpose(2, 4, 0, 1, 3).reshape(BATCH, SEQ, EMBED_DIM)
